# baseline (speedup 1.0000x reference)
import functools

import jax
import jax.numpy as jnp
from jax import lax
from jax.experimental import pallas as pl
from jax.experimental.pallas import tpu as pltpu
from jax.experimental.pallas import tpu_sc as plsc

N = 10000
D = 128
E = 320000
HF = 64
NC = 2
NS = 16
LN = 16
CH = 64
NCHT = E // CH
WT = 10
RPT = N // WT
ZR = 40
NB = 1000
GRID = N // NB


def _stage1_body(ft, fs, wnt, wa1, wa2, whh, bnt, batt, bhh,
                 src_o, dst_o, ght_o, mxacc, mxo):
    i = pl.program_id(0)
    t_tok = jnp.dot(ft[...], wnt[...], preferred_element_type=jnp.float32) + bnt[...]
    t_srl = jnp.dot(fs[...], wnt[...], preferred_element_type=jnp.float32) + bnt[...]
    a_tok = jnp.dot(t_tok, wa1[...], preferred_element_type=jnp.float32)
    a_srl = jnp.dot(t_srl, wa2[...], preferred_element_type=jnp.float32) + batt[...]
    gh = jnp.dot(ft[...], whh[...], preferred_element_type=jnp.float32) + bhh[...]
    ftv = ft[...]
    src_o[0] = jnp.concatenate([a_tok[:, :HF], t_tok[:, :HF]], axis=1)
    src_o[1] = jnp.concatenate([a_tok[:, HF:], t_tok[:, HF:]], axis=1)
    dst_o[...] = a_srl
    ght_o[0] = jnp.concatenate(
        [gh[:, 0:HF], gh[:, D:D + HF], gh[:, 2 * D:2 * D + HF], ftv[:, :HF]], axis=1)
    ght_o[1] = jnp.concatenate(
        [gh[:, HF:D], gh[:, D + HF:2 * D], gh[:, 2 * D + HF:3 * D], ftv[:, HF:]], axis=1)
    bm = jnp.stack([jnp.max(a_tok, axis=0), jnp.max(a_srl, axis=0)], axis=0)

    @pl.when(i == 0)
    def _():
        mxacc[...] = bm

    @pl.when(i > 0)
    def _():
        mxacc[...] = jnp.maximum(mxacc[...], bm)

    @pl.when(i == pl.num_programs(0) - 1)
    def _():
        m = mxacc[0:1, :] + mxacc[1:2, :]
        mxo[...] = jnp.maximum(m, 0.01 * m)


def _stage1(ft, fs, wnt, wa1, wa2, whh, bnt, batt, bhh):
    f32 = jnp.float32
    return pl.pallas_call(
        _stage1_body,
        grid=(GRID,),
        in_specs=[
            pl.BlockSpec((NB, D), lambda i: (i, 0)),
            pl.BlockSpec((NB, D), lambda i: (i, 0)),
            pl.BlockSpec((D, D), lambda i: (0, 0)),
            pl.BlockSpec((D, D), lambda i: (0, 0)),
            pl.BlockSpec((D, D), lambda i: (0, 0)),
            pl.BlockSpec((D, 3 * D), lambda i: (0, 0)),
            pl.BlockSpec((1, D), lambda i: (0, 0)),
            pl.BlockSpec((1, D), lambda i: (0, 0)),
            pl.BlockSpec((1, 3 * D), lambda i: (0, 0)),
        ],
        out_specs=[
            pl.BlockSpec((NC, NB, 2 * HF), lambda i: (0, i, 0)),
            pl.BlockSpec((NB, D), lambda i: (i, 0)),
            pl.BlockSpec((NC, NB, 4 * HF), lambda i: (0, i, 0)),
            pl.BlockSpec((2, D), lambda i: (0, 0)),
            pl.BlockSpec((1, D), lambda i: (0, 0)),
        ],
        out_shape=[
            jax.ShapeDtypeStruct((NC, N, 2 * HF), f32),
            jax.ShapeDtypeStruct((N, D), f32),
            jax.ShapeDtypeStruct((NC, N, 4 * HF), f32),
            jax.ShapeDtypeStruct((2, D), f32),
            jax.ShapeDtypeStruct((1, D), f32),
        ],
    )(ft, fs, wnt, wa1, wa2, whh, bnt, batt, bhh)


def _pass_a(src_tab, dst_tab, mx, idx2):
    f32 = jnp.float32
    mesh = plsc.VectorSubcoreMesh(core_axis_name="c", subcore_axis_name="s")

    @functools.partial(
        pl.kernel,
        out_type=jax.ShapeDtypeStruct((NC * N, 2 * HF), f32),
        mesh=mesh,
        scratch_types=[
            pltpu.VMEM_SHARED((N, 2 * HF), f32),
            pltpu.VMEM((2, CH), jnp.int32),
            pltpu.VMEM((2, CH), jnp.int32),
            pltpu.VMEM((CH, 2 * HF), f32),
            pltpu.VMEM((CH, 2 * HF), f32),
            pltpu.VMEM((CH, D), f32),
            pltpu.VMEM((CH, D), f32),
            pltpu.VMEM((CH, 2 * HF), f32),
            pltpu.VMEM((CH, 2 * HF), f32),
            pltpu.VMEM((HF,), f32),
            pltpu.SemaphoreType.DMA,
            pltpu.SemaphoreType.DMA,
            pltpu.SemaphoreType.DMA,
            pltpu.SemaphoreType.DMA,
            pltpu.SemaphoreType.DMA,
            pltpu.SemaphoreType.DMA,
        ],
    )
    def k(src_h, dst_h, mx_h, idx2_h, out_h,
          acc, ib0, ib1, sr0, sr1, dr0, dr1, ct0, ct1, mxv,
          ss0, ss1, sd0, sd1, sc0, sc1):
        c = lax.axis_index("c")
        w = lax.axis_index("s")
        zero16 = jnp.zeros((LN,), f32)
        off = c * N
        nc = jnp.where(w < NCHT % NS, NCHT // NS + 1, NCHT // NS)

        def zrow(r, _):
            for j in range(2 * HF // LN):
                ct0[r, pl.ds(LN * j, LN)] = zero16
            return 0

        lax.fori_loop(0, ZR, zrow, 0)

        @pl.when(w < WT)
        def _():
            for z in range(RPT // ZR):
                pltpu.sync_copy(ct0.at[pl.ds(0, ZR)],
                                acc.at[pl.ds(w * RPT + z * ZR, ZR)])

        plsc.subcore_barrier()
        pltpu.sync_copy(mx_h.at[pl.ds(c * HF, HF)], mxv)

        def issue(t, ib, sr, dr, ct, sems):
            @pl.when(t >= 2)
            def _():
                pltpu.make_async_copy(ct, acc.at[ib.at[1]], sems[2]).wait()

            pltpu.sync_copy(idx2_h.at[w + t * NS], ib)
            for j in range(CH // LN):
                sl = pl.ds(LN * j, LN)
                ib[0, sl] = ib[0, sl] + off
            pltpu.async_copy(src_h.at[ib.at[0]], sr, sems[0])
            pltpu.async_copy(dst_h.at[ib.at[1]], dr, sems[1])

        def work(ib, sr, dr, ct, sems):
            pltpu.make_async_copy(src_h.at[ib.at[0]], sr, sems[0]).wait()
            pltpu.make_async_copy(dst_h.at[ib.at[1]], dr, sems[1]).wait()
            dbase = c * HF

            @plsc.parallel_loop(0, CH, unroll=4)
            def _edge(i):
                for j in range(HF // LN):
                    sl = pl.ds(LN * j, LN)
                    sl2 = pl.ds(HF + LN * j, LN)
                    v = sr[i, sl] + dr[i, pl.ds(dbase + LN * j, LN)]
                    ee = jnp.maximum(v, 0.01 * v)
                    ex = jnp.exp(ee - mxv[sl])
                    ct[i, sl] = ex * sr[i, sl2]
                    ct[i, sl2] = ex
            pltpu.async_copy(ct, acc.at[ib.at[1]], sems[2], add=True)

        @pl.when(nc > 0)
        def _():
            issue(0, ib0, sr0, dr0, ct0, (ss0, sd0, sc0))

        def body(t, _):
            @pl.when((t & 1) == 0)
            def _():
                @pl.when(t + 1 < nc)
                def _():
                    issue(t + 1, ib1, sr1, dr1, ct1, (ss1, sd1, sc1))
                work(ib0, sr0, dr0, ct0, (ss0, sd0, sc0))

            @pl.when((t & 1) == 1)
            def _():
                @pl.when(t + 1 < nc)
                def _():
                    issue(t + 1, ib0, sr0, dr0, ct0, (ss0, sd0, sc0))
                work(ib1, sr1, dr1, ct1, (ss1, sd1, sc1))

            return 0

        lax.fori_loop(0, nc, body, 0)
        pltpu.make_async_copy(ct0, acc.at[ib0.at[1]], sc0).wait()
        pltpu.make_async_copy(ct1, acc.at[ib1.at[1]], sc1).wait()
        plsc.subcore_barrier()

        @pl.when(w < WT)
        def _():
            for z in range(RPT // ZR):
                sl = pl.ds(w * RPT + z * ZR, ZR)
                pltpu.sync_copy(acc.at[sl], ct0.at[pl.ds(0, ZR)])
                pltpu.sync_copy(ct0.at[pl.ds(0, ZR)],
                                out_h.at[pl.ds(c * N + w * RPT + z * ZR, ZR)])

    return k(src_tab, dst_tab, mx, idx2)


def _stage3_body(accp, fs, wih, bih, hsrl_o, git_o):
    numer = jnp.concatenate([accp[0, :, :HF], accp[1, :, :HF]], axis=1)
    denom = jnp.concatenate([accp[0, :, HF:], accp[1, :, HF:]], axis=1)
    fsv = fs[...]
    h = jnp.where(denom > 0.0, numer / denom, fsv)
    hsrl_o[...] = h
    gi = jnp.dot(h, wih[...], preferred_element_type=jnp.float32) + bih[...]
    pad = jnp.zeros((NB, HF), jnp.float32)
    git_o[0] = jnp.concatenate(
        [gi[:, 0:HF], gi[:, D:D + HF], gi[:, 2 * D:2 * D + HF], pad], axis=1)
    git_o[1] = jnp.concatenate(
        [gi[:, HF:D], gi[:, D + HF:2 * D], gi[:, 2 * D + HF:3 * D], pad], axis=1)


def _stage3(acc_a, fs, wih, bih):
    f32 = jnp.float32
    return pl.pallas_call(
        _stage3_body,
        grid=(GRID,),
        in_specs=[
            pl.BlockSpec((NC, NB, 2 * HF), lambda i: (0, i, 0)),
            pl.BlockSpec((NB, D), lambda i: (i, 0)),
            pl.BlockSpec((D, 3 * D), lambda i: (0, 0)),
            pl.BlockSpec((1, 3 * D), lambda i: (0, 0)),
        ],
        out_specs=[
            pl.BlockSpec((NB, D), lambda i: (i, 0)),
            pl.BlockSpec((NC, NB, 4 * HF), lambda i: (0, i, 0)),
        ],
        out_shape=[
            jax.ShapeDtypeStruct((N, D), f32),
            jax.ShapeDtypeStruct((NC, N, 4 * HF), f32),
        ],
    )(acc_a, fs, wih, bih)


def _pass_b(git_tab, ght_tab, idx2):
    f32 = jnp.float32
    mesh = plsc.VectorSubcoreMesh(core_axis_name="c", subcore_axis_name="s")
    HN = N // 2
    WT2 = 5
    RPT2 = HN // WT2

    @functools.partial(
        pl.kernel,
        out_type=jax.ShapeDtypeStruct((NC * HN, 2 * HF), f32),
        mesh=mesh,
        scratch_types=[
            pltpu.VMEM_SHARED((HN, 2 * HF), f32),
            pltpu.VMEM((2, CH), jnp.int32),
            pltpu.VMEM((2, CH), jnp.int32),
            pltpu.VMEM((CH,), jnp.int32),
            pltpu.VMEM((CH,), jnp.int32),
            pltpu.VMEM((CH + LN,), jnp.int32),
            pltpu.VMEM((CH + LN,), jnp.int32),
            pltpu.VMEM((CH, 4 * HF), f32),
            pltpu.VMEM((CH, 4 * HF), f32),
            pltpu.VMEM((CH, 4 * HF), f32),
            pltpu.VMEM((CH, 4 * HF), f32),
            pltpu.VMEM((CH, 2 * HF), f32),
            pltpu.VMEM((CH, 2 * HF), f32),
            pltpu.SemaphoreType.DMA,
            pltpu.SemaphoreType.DMA,
            pltpu.SemaphoreType.DMA,
            pltpu.SemaphoreType.DMA,
            pltpu.SemaphoreType.DMA,
            pltpu.SemaphoreType.DMA,
        ],
    )
    def k(git_h, ght_h, idx2_h, out_h,
          acc, ib0, ib1, ih0, ih1, ip0, ip1, gr0, gr1, hr0, hr1, ct0, ct1,
          ss0, ss1, sd0, sd1, sc0, sc1):
        c = lax.axis_index("c")
        w = lax.axis_index("s")
        one = jnp.full((LN,), 1.0, f32)
        two = jnp.full((LN,), 2.0, f32)
        zero16 = jnp.zeros((LN,), f32)
        off = c * N
        nc = jnp.where(w < NCHT % NS, NCHT // NS + 1, NCHT // NS)

        def zrow(r, _):
            for j in range(2 * HF // LN):
                ct0[r, pl.ds(LN * j, LN)] = zero16
            return 0

        lax.fori_loop(0, ZR, zrow, 0)

        @pl.when(w < WT2)
        def _():
            for z in range(RPT2 // ZR):
                pltpu.sync_copy(ct0.at[pl.ds(0, ZR)],
                                acc.at[pl.ds(w * RPT2 + z * ZR, ZR)])

        plsc.subcore_barrier()

        def issue(t, ib, ih, ip, gr, hr, ct, sems):
            @pl.when(t >= 2)
            def _():
                pltpu.make_async_copy(ct, acc.at[ih], sems[2]).wait()

            pltpu.sync_copy(idx2_h.at[w + t * NS], ib)
            for j in range(CH // LN):
                sl = pl.ds(LN * j, LN)
                dv = ib[1, sl]
                ip[sl] = dv
                ih[sl] = lax.shift_right_logical(dv, 1)
                ib[0, sl] = ib[0, sl] + off
                ib[1, sl] = dv + off
            pltpu.async_copy(git_h.at[ib.at[0]], gr, sems[0])
            pltpu.async_copy(ght_h.at[ib.at[1]], hr, sems[1])

        def work(ib, ih, ip, gr, hr, ct, sems):
            pltpu.make_async_copy(git_h.at[ib.at[0]], gr, sems[0]).wait()
            pltpu.make_async_copy(ght_h.at[ib.at[1]], hr, sems[1]).wait()

            @plsc.parallel_loop(0, CH, unroll=6)
            def _edge(i):
                d = ip[pl.ds(i, LN)][0]
                even = (d & 1) == 0
                for j in range(HF // LN):
                    s0 = pl.ds(LN * j, LN)
                    s1 = pl.ds(HF + LN * j, LN)
                    s2 = pl.ds(2 * HF + LN * j, LN)
                    s3 = pl.ds(3 * HF + LN * j, LN)
                    r = one / (one + jnp.exp(-(gr[i, s0] + hr[i, s0])))
                    zg = one / (one + jnp.exp(-(gr[i, s1] + hr[i, s1])))
                    a2 = gr[i, s2] + r * hr[i, s2]
                    n = one - two / (jnp.exp(a2 + a2) + one)
                    m = (one - zg) * n + zg * hr[i, s3]
                    ct[i, s0] = jnp.where(even, m, 0.0)
                    ct[i, s1] = jnp.where(even, 0.0, m)
            pltpu.async_copy(ct, acc.at[ih], sems[2], add=True)

        @pl.when(nc > 0)
        def _():
            issue(0, ib0, ih0, ip0, gr0, hr0, ct0, (ss0, sd0, sc0))

        def body(t, _):
            @pl.when((t & 1) == 0)
            def _():
                @pl.when(t + 1 < nc)
                def _():
                    issue(t + 1, ib1, ih1, ip1, gr1, hr1, ct1, (ss1, sd1, sc1))
                work(ib0, ih0, ip0, gr0, hr0, ct0, (ss0, sd0, sc0))

            @pl.when((t & 1) == 1)
            def _():
                @pl.when(t + 1 < nc)
                def _():
                    issue(t + 1, ib0, ih0, ip0, gr0, hr0, ct0, (ss0, sd0, sc0))
                work(ib1, ih1, ip1, gr1, hr1, ct1, (ss1, sd1, sc1))

            return 0

        lax.fori_loop(0, nc, body, 0)
        pltpu.make_async_copy(ct0, acc.at[ih0], sc0).wait()
        pltpu.make_async_copy(ct1, acc.at[ih1], sc1).wait()
        plsc.subcore_barrier()

        @pl.when(w < WT2)
        def _():
            for z in range(RPT2 // ZR):
                sl = pl.ds(w * RPT2 + z * ZR, ZR)
                pltpu.sync_copy(acc.at[sl], ct0.at[pl.ds(0, ZR)])
                pltpu.sync_copy(ct0.at[pl.ds(0, ZR)],
                                out_h.at[pl.ds(c * HN + w * RPT2 + z * ZR, ZR)])

    return k(git_tab, ght_tab, idx2)


def _stage5_body(hat, wih, whh, bih, bhh, out):
    h = hat[...]
    gi = jnp.dot(h, wih[...], preferred_element_type=jnp.float32) + bih[...]
    gh = jnp.dot(h, whh[...], preferred_element_type=jnp.float32) + bhh[...]
    r = jax.nn.sigmoid(gi[:, :D] + gh[:, :D])
    z = jax.nn.sigmoid(gi[:, D:2 * D] + gh[:, D:2 * D])
    n = jnp.tanh(gi[:, 2 * D:] + r * gh[:, 2 * D:])
    out[...] = (1.0 - z) * n + z * h


def _stage5(hat, wih, whh, bih, bhh):
    return pl.pallas_call(
        _stage5_body,
        grid=(GRID,),
        in_specs=[
            pl.BlockSpec((NB, D), lambda i: (i, 0)),
            pl.BlockSpec((D, 3 * D), lambda i: (0, 0)),
            pl.BlockSpec((D, 3 * D), lambda i: (0, 0)),
            pl.BlockSpec((1, 3 * D), lambda i: (0, 0)),
            pl.BlockSpec((1, 3 * D), lambda i: (0, 0)),
        ],
        out_specs=pl.BlockSpec((NB, D), lambda i: (i, 0)),
        out_shape=jax.ShapeDtypeStruct((N, D), jnp.float32),
    )(hat, wih, whh, bih, bhh)


def kernel(feat_tok, feat_srl, W_node_trans, b_node_trans, W_node_att,
           b_node_att, W_ih, W_hh, b_ih, b_hh, edge_tok2srl, edge_srl2tok):
    f32 = jnp.float32
    ft = feat_tok.astype(f32)
    fs = feat_srl.astype(f32)
    wnt = W_node_trans.T.astype(f32)
    wa1 = W_node_att[:, :D].T.astype(f32)
    wa2 = W_node_att[:, D:].T.astype(f32)
    wih = W_ih.T.astype(f32)
    whh = W_hh.T.astype(f32)
    bnt = b_node_trans.reshape(1, D).astype(f32)
    batt = b_node_att.reshape(1, D).astype(f32)
    bih = b_ih.reshape(1, 3 * D).astype(f32)
    bhh = b_hh.reshape(1, 3 * D).astype(f32)
    idx2_a = edge_tok2srl.astype(jnp.int32).reshape(2, NCHT, CH)
    idx2_a = jnp.swapaxes(idx2_a, 0, 1)
    idx2_b = edge_srl2tok.astype(jnp.int32).reshape(2, NCHT, CH)
    idx2_b = jnp.swapaxes(idx2_b, 0, 1)

    src_tab, dst_tab, ght_tab, _, mxo = _stage1(
        ft, fs, wnt, wa1, wa2, whh, bnt, batt, bhh)
    acc_a = _pass_a(src_tab.reshape(NC * N, 2 * HF), dst_tab,
                    mxo.reshape(D), idx2_a)
    h_srl, git_tab = _stage3(acc_a.reshape(NC, N, 2 * HF), fs, wih, bih)
    acc_b = _pass_b(git_tab.reshape(NC * N, 4 * HF),
                    ght_tab.reshape(NC * N, 4 * HF), idx2_b)
    acc_b = acc_b.reshape(NC, N, HF)
    h_at = jnp.concatenate([acc_b[0], acc_b[1]], axis=1)
    h_tok = _stage5(h_at, wih, whh, bih, bhh)
    return (h_tok, h_srl)


# speedup vs baseline: 2.5080x; 2.5080x over previous
import functools

import jax
import jax.numpy as jnp
from jax import lax
from jax.experimental import pallas as pl
from jax.experimental.pallas import tpu as pltpu
from jax.experimental.pallas import tpu_sc as plsc

N = 10000
D = 128
E = 320000
HF = 64
NC = 2
NS = 16
LN = 16
CH = 64
NCHT = E // CH
WT = 10
RPT = N // WT
ZR = 40
NB = 1000
GRID = N // NB


def _stage1_body(ft, fs, wnt, wa1, wa2, whh, bnt, batt, bhh,
                 src_o, dst_o, ght_o, mxacc, mxo):
    i = pl.program_id(0)
    t_tok = jnp.dot(ft[...], wnt[...], preferred_element_type=jnp.float32) + bnt[...]
    t_srl = jnp.dot(fs[...], wnt[...], preferred_element_type=jnp.float32) + bnt[...]
    a_tok = jnp.dot(t_tok, wa1[...], preferred_element_type=jnp.float32)
    a_srl = jnp.dot(t_srl, wa2[...], preferred_element_type=jnp.float32) + batt[...]
    gh = jnp.dot(ft[...], whh[...], preferred_element_type=jnp.float32) + bhh[...]
    ftv = ft[...]
    src_o[0] = jnp.concatenate([a_tok[:, :HF], t_tok[:, :HF]], axis=1)
    src_o[1] = jnp.concatenate([a_tok[:, HF:], t_tok[:, HF:]], axis=1)
    dst_o[...] = a_srl
    ght_o[0] = jnp.concatenate(
        [gh[:, 0:HF], gh[:, D:D + HF], gh[:, 2 * D:2 * D + HF], ftv[:, :HF]], axis=1)
    ght_o[1] = jnp.concatenate(
        [gh[:, HF:D], gh[:, D + HF:2 * D], gh[:, 2 * D + HF:3 * D], ftv[:, HF:]], axis=1)
    bm = jnp.stack([jnp.max(a_tok, axis=0), jnp.max(a_srl, axis=0)], axis=0)

    @pl.when(i == 0)
    def _():
        mxacc[...] = bm

    @pl.when(i > 0)
    def _():
        mxacc[...] = jnp.maximum(mxacc[...], bm)

    @pl.when(i == pl.num_programs(0) - 1)
    def _():
        m = mxacc[0:1, :] + mxacc[1:2, :]
        mxo[...] = jnp.maximum(m, 0.01 * m)


def _stage1(ft, fs, wnt, wa1, wa2, whh, bnt, batt, bhh):
    f32 = jnp.float32
    return pl.pallas_call(
        _stage1_body,
        grid=(GRID,),
        in_specs=[
            pl.BlockSpec((NB, D), lambda i: (i, 0)),
            pl.BlockSpec((NB, D), lambda i: (i, 0)),
            pl.BlockSpec((D, D), lambda i: (0, 0)),
            pl.BlockSpec((D, D), lambda i: (0, 0)),
            pl.BlockSpec((D, D), lambda i: (0, 0)),
            pl.BlockSpec((D, 3 * D), lambda i: (0, 0)),
            pl.BlockSpec((1, D), lambda i: (0, 0)),
            pl.BlockSpec((1, D), lambda i: (0, 0)),
            pl.BlockSpec((1, 3 * D), lambda i: (0, 0)),
        ],
        out_specs=[
            pl.BlockSpec((NC, NB, 2 * HF), lambda i: (0, i, 0)),
            pl.BlockSpec((NB, D), lambda i: (i, 0)),
            pl.BlockSpec((NC, NB, 4 * HF), lambda i: (0, i, 0)),
            pl.BlockSpec((2, D), lambda i: (0, 0)),
            pl.BlockSpec((1, D), lambda i: (0, 0)),
        ],
        out_shape=[
            jax.ShapeDtypeStruct((NC, N, 2 * HF), f32),
            jax.ShapeDtypeStruct((N, D), f32),
            jax.ShapeDtypeStruct((NC, N, 4 * HF), f32),
            jax.ShapeDtypeStruct((2, D), f32),
            jax.ShapeDtypeStruct((1, D), f32),
        ],
    )(ft, fs, wnt, wa1, wa2, whh, bnt, batt, bhh)


def _pass_a(src_tab, dst_tab, mx, idx2):
    f32 = jnp.float32
    mesh = plsc.VectorSubcoreMesh(core_axis_name="c", subcore_axis_name="s")

    @functools.partial(
        pl.kernel,
        out_type=jax.ShapeDtypeStruct((NC * N, 2 * HF), f32),
        mesh=mesh,
        scratch_types=[
            pltpu.VMEM_SHARED((N, 2 * HF), f32),
            pltpu.VMEM((2, CH), jnp.int32),
            pltpu.VMEM((2, CH), jnp.int32),
            pltpu.VMEM((CH, 2 * HF), f32),
            pltpu.VMEM((CH, 2 * HF), f32),
            pltpu.VMEM((CH, D), f32),
            pltpu.VMEM((CH, D), f32),
            pltpu.VMEM((CH, 2 * HF), f32),
            pltpu.VMEM((CH, 2 * HF), f32),
            pltpu.VMEM((HF,), f32),
            pltpu.SemaphoreType.DMA,
            pltpu.SemaphoreType.DMA,
            pltpu.SemaphoreType.DMA,
            pltpu.SemaphoreType.DMA,
            pltpu.SemaphoreType.DMA,
            pltpu.SemaphoreType.DMA,
        ],
    )
    def k(src_h, dst_h, mx_h, idx2_h, out_h,
          acc, ib0, ib1, sr0, sr1, dr0, dr1, ct0, ct1, mxv,
          ss0, ss1, sd0, sd1, sc0, sc1):
        c = lax.axis_index("c")
        w = lax.axis_index("s")
        zero16 = jnp.zeros((LN,), f32)
        off = c * N
        nc = jnp.where(w < NCHT % NS, NCHT // NS + 1, NCHT // NS)

        def zrow(r, _):
            for j in range(2 * HF // LN):
                ct0[r, pl.ds(LN * j, LN)] = zero16
            return 0

        lax.fori_loop(0, ZR, zrow, 0)

        @pl.when(w < WT)
        def _():
            for z in range(RPT // ZR):
                pltpu.sync_copy(ct0.at[pl.ds(0, ZR)],
                                acc.at[pl.ds(w * RPT + z * ZR, ZR)])

        plsc.subcore_barrier()
        pltpu.sync_copy(mx_h.at[pl.ds(c * HF, HF)], mxv)

        def issue(t, ib, sr, dr, ct, sems):
            @pl.when(t >= 2)
            def _():
                pltpu.make_async_copy(ct, acc.at[ib.at[1]], sems[2]).wait()

            pltpu.sync_copy(idx2_h.at[w + t * NS], ib)
            for j in range(CH // LN):
                sl = pl.ds(LN * j, LN)
                ib[0, sl] = ib[0, sl] + off
            pltpu.async_copy(src_h.at[ib.at[0]], sr, sems[0])
            pltpu.async_copy(dst_h.at[ib.at[1]], dr, sems[1])

        def work(ib, sr, dr, ct, sems):
            pltpu.make_async_copy(src_h.at[ib.at[0]], sr, sems[0]).wait()
            pltpu.make_async_copy(dst_h.at[ib.at[1]], dr, sems[1]).wait()
            dbase = c * HF

            @plsc.parallel_loop(0, CH, unroll=4)
            def _edge(i):
                for j in range(HF // LN):
                    sl = pl.ds(LN * j, LN)
                    sl2 = pl.ds(HF + LN * j, LN)
                    v = sr[i, sl] + dr[i, pl.ds(dbase + LN * j, LN)]
                    ee = jnp.maximum(v, 0.01 * v)
                    ex = jnp.exp(ee - mxv[sl])
                    ct[i, sl] = ex * sr[i, sl2]
                    ct[i, sl2] = ex
            pltpu.async_copy(ct, acc.at[ib.at[1]], sems[2], add=True)

        @pl.when(nc > 0)
        def _():
            issue(0, ib0, sr0, dr0, ct0, (ss0, sd0, sc0))

        def body(t, _):
            @pl.when((t & 1) == 0)
            def _():
                @pl.when(t + 1 < nc)
                def _():
                    issue(t + 1, ib1, sr1, dr1, ct1, (ss1, sd1, sc1))
                work(ib0, sr0, dr0, ct0, (ss0, sd0, sc0))

            @pl.when((t & 1) == 1)
            def _():
                @pl.when(t + 1 < nc)
                def _():
                    issue(t + 1, ib0, sr0, dr0, ct0, (ss0, sd0, sc0))
                work(ib1, sr1, dr1, ct1, (ss1, sd1, sc1))

            return 0

        lax.fori_loop(0, nc, body, 0)
        pltpu.make_async_copy(ct0, acc.at[ib0.at[1]], sc0).wait()
        pltpu.make_async_copy(ct1, acc.at[ib1.at[1]], sc1).wait()
        plsc.subcore_barrier()

        @pl.when(w < WT)
        def _():
            for z in range(RPT // ZR):
                sl = pl.ds(w * RPT + z * ZR, ZR)
                pltpu.sync_copy(acc.at[sl], ct0.at[pl.ds(0, ZR)])
                pltpu.sync_copy(ct0.at[pl.ds(0, ZR)],
                                out_h.at[pl.ds(c * N + w * RPT + z * ZR, ZR)])

    return k(src_tab, dst_tab, mx, idx2)


def _stage3_body(accp, fs, wih, bih, hsrl_o, git_o):
    numer = jnp.concatenate([accp[0, :, :HF], accp[1, :, :HF]], axis=1)
    denom = jnp.concatenate([accp[0, :, HF:], accp[1, :, HF:]], axis=1)
    fsv = fs[...]
    h = jnp.where(denom > 0.0, numer / denom, fsv)
    hsrl_o[...] = h
    gi = jnp.dot(h, wih[...], preferred_element_type=jnp.float32) + bih[...]
    pad = jnp.zeros((NB, HF), jnp.float32)
    git_o[0] = jnp.concatenate(
        [gi[:, 0:HF], gi[:, D:D + HF], gi[:, 2 * D:2 * D + HF], pad], axis=1)
    git_o[1] = jnp.concatenate(
        [gi[:, HF:D], gi[:, D + HF:2 * D], gi[:, 2 * D + HF:3 * D], pad], axis=1)


def _stage3(acc_a, fs, wih, bih):
    f32 = jnp.float32
    return pl.pallas_call(
        _stage3_body,
        grid=(GRID,),
        in_specs=[
            pl.BlockSpec((NC, NB, 2 * HF), lambda i: (0, i, 0)),
            pl.BlockSpec((NB, D), lambda i: (i, 0)),
            pl.BlockSpec((D, 3 * D), lambda i: (0, 0)),
            pl.BlockSpec((1, 3 * D), lambda i: (0, 0)),
        ],
        out_specs=[
            pl.BlockSpec((NB, D), lambda i: (i, 0)),
            pl.BlockSpec((NC, NB, 4 * HF), lambda i: (0, i, 0)),
        ],
        out_shape=[
            jax.ShapeDtypeStruct((N, D), f32),
            jax.ShapeDtypeStruct((NC, N, 4 * HF), f32),
        ],
    )(acc_a, fs, wih, bih)


def _pass_b(git_tab, ght_tab, idx2):
    f32 = jnp.float32
    mesh = plsc.VectorSubcoreMesh(core_axis_name="c", subcore_axis_name="s")
    HN = N // 2
    WT2 = 5
    RPT2 = HN // WT2

    @functools.partial(
        pl.kernel,
        out_type=jax.ShapeDtypeStruct((NC * HN, 2 * HF), f32),
        mesh=mesh,
        scratch_types=[
            pltpu.VMEM_SHARED((HN, 2 * HF), f32),
            pltpu.VMEM((2, CH), jnp.int32),
            pltpu.VMEM((2, CH), jnp.int32),
            pltpu.VMEM((CH,), jnp.int32),
            pltpu.VMEM((CH,), jnp.int32),
            pltpu.VMEM((CH + LN,), jnp.int32),
            pltpu.VMEM((CH + LN,), jnp.int32),
            pltpu.VMEM((CH, 4 * HF), f32),
            pltpu.VMEM((CH, 4 * HF), f32),
            pltpu.VMEM((CH, 4 * HF), f32),
            pltpu.VMEM((CH, 4 * HF), f32),
            pltpu.VMEM((CH, 2 * HF), f32),
            pltpu.VMEM((CH, 2 * HF), f32),
            pltpu.SemaphoreType.DMA,
            pltpu.SemaphoreType.DMA,
            pltpu.SemaphoreType.DMA,
            pltpu.SemaphoreType.DMA,
            pltpu.SemaphoreType.DMA,
            pltpu.SemaphoreType.DMA,
        ],
    )
    def k(git_h, ght_h, idx2_h, out_h,
          acc, ib0, ib1, ih0, ih1, ip0, ip1, gr0, gr1, hr0, hr1, ct0, ct1,
          ss0, ss1, sd0, sd1, sc0, sc1):
        c = lax.axis_index("c")
        w = lax.axis_index("s")
        one = jnp.full((LN,), 1.0, f32)
        two = jnp.full((LN,), 2.0, f32)
        zero16 = jnp.zeros((LN,), f32)
        off = c * N
        nc = jnp.where(w < NCHT % NS, NCHT // NS + 1, NCHT // NS)

        def zrow(r, _):
            for j in range(2 * HF // LN):
                ct0[r, pl.ds(LN * j, LN)] = zero16
            return 0

        lax.fori_loop(0, ZR, zrow, 0)

        @pl.when(w < WT2)
        def _():
            for z in range(RPT2 // ZR):
                pltpu.sync_copy(ct0.at[pl.ds(0, ZR)],
                                acc.at[pl.ds(w * RPT2 + z * ZR, ZR)])

        plsc.subcore_barrier()

        def issue(t, ib, ih, ip, gr, hr, ct, sems):
            @pl.when(t >= 2)
            def _():
                pltpu.make_async_copy(ct, acc.at[ih], sems[2]).wait()

            pltpu.sync_copy(idx2_h.at[w + t * NS], ib)
            for j in range(CH // LN):
                sl = pl.ds(LN * j, LN)
                dv = ib[1, sl]
                ip[sl] = dv
                ih[sl] = lax.shift_right_logical(dv, 1)
                ib[0, sl] = ib[0, sl] + off
                ib[1, sl] = dv + off
            pltpu.async_copy(git_h.at[ib.at[0]], gr, sems[0])
            pltpu.async_copy(ght_h.at[ib.at[1]], hr, sems[1])

        def work(ib, ih, ip, gr, hr, ct, sems):
            pltpu.make_async_copy(git_h.at[ib.at[0]], gr, sems[0]).wait()
            pltpu.make_async_copy(ght_h.at[ib.at[1]], hr, sems[1]).wait()

            @plsc.parallel_loop(0, CH, unroll=4)
            def _edge(i):
                d = ip[pl.ds(i, LN)][0]
                even = (d & 1) == 0
                for j in range(HF // LN):
                    s0 = pl.ds(LN * j, LN)
                    s1 = pl.ds(HF + LN * j, LN)
                    s2 = pl.ds(2 * HF + LN * j, LN)
                    s3 = pl.ds(3 * HF + LN * j, LN)
                    r = one / (one + jnp.exp(-(gr[i, s0] + hr[i, s0])))
                    zg = one / (one + jnp.exp(-(gr[i, s1] + hr[i, s1])))
                    a2 = gr[i, s2] + r * hr[i, s2]
                    n = one - two / (jnp.exp(a2 + a2) + one)
                    m = (one - zg) * n + zg * hr[i, s3]
                    ct[i, s0] = jnp.where(even, m, 0.0)
                    ct[i, s1] = jnp.where(even, 0.0, m)
            pltpu.async_copy(ct, acc.at[ih], sems[2], add=True)

        @pl.when(nc > 0)
        def _():
            issue(0, ib0, ih0, ip0, gr0, hr0, ct0, (ss0, sd0, sc0))

        def body(t, _):
            @pl.when((t & 1) == 0)
            def _():
                @pl.when(t + 1 < nc)
                def _():
                    issue(t + 1, ib1, ih1, ip1, gr1, hr1, ct1, (ss1, sd1, sc1))
                work(ib0, ih0, ip0, gr0, hr0, ct0, (ss0, sd0, sc0))

            @pl.when((t & 1) == 1)
            def _():
                @pl.when(t + 1 < nc)
                def _():
                    issue(t + 1, ib0, ih0, ip0, gr0, hr0, ct0, (ss0, sd0, sc0))
                work(ib1, ih1, ip1, gr1, hr1, ct1, (ss1, sd1, sc1))

            return 0

        lax.fori_loop(0, nc, body, 0)
        pltpu.make_async_copy(ct0, acc.at[ih0], sc0).wait()
        pltpu.make_async_copy(ct1, acc.at[ih1], sc1).wait()
        plsc.subcore_barrier()

        @pl.when(w < WT2)
        def _():
            for z in range(RPT2 // ZR):
                sl = pl.ds(w * RPT2 + z * ZR, ZR)
                pltpu.sync_copy(acc.at[sl], ct0.at[pl.ds(0, ZR)])
                pltpu.sync_copy(ct0.at[pl.ds(0, ZR)],
                                out_h.at[pl.ds(c * HN + w * RPT2 + z * ZR, ZR)])

    return k(git_tab, ght_tab, idx2)


def _stage5_body(hat, wih, whh, bih, bhh, out):
    h = hat[...]
    gi = jnp.dot(h, wih[...], preferred_element_type=jnp.float32) + bih[...]
    gh = jnp.dot(h, whh[...], preferred_element_type=jnp.float32) + bhh[...]
    r = jax.nn.sigmoid(gi[:, :D] + gh[:, :D])
    z = jax.nn.sigmoid(gi[:, D:2 * D] + gh[:, D:2 * D])
    n = jnp.tanh(gi[:, 2 * D:] + r * gh[:, 2 * D:])
    out[...] = (1.0 - z) * n + z * h


def _stage5(hat, wih, whh, bih, bhh):
    return pl.pallas_call(
        _stage5_body,
        grid=(GRID,),
        in_specs=[
            pl.BlockSpec((NB, D), lambda i: (i, 0)),
            pl.BlockSpec((D, 3 * D), lambda i: (0, 0)),
            pl.BlockSpec((D, 3 * D), lambda i: (0, 0)),
            pl.BlockSpec((1, 3 * D), lambda i: (0, 0)),
            pl.BlockSpec((1, 3 * D), lambda i: (0, 0)),
        ],
        out_specs=pl.BlockSpec((NB, D), lambda i: (i, 0)),
        out_shape=jax.ShapeDtypeStruct((N, D), jnp.float32),
    )(hat, wih, whh, bih, bhh)


def kernel(feat_tok, feat_srl, W_node_trans, b_node_trans, W_node_att,
           b_node_att, W_ih, W_hh, b_ih, b_hh, edge_tok2srl, edge_srl2tok):
    f32 = jnp.float32
    ft = feat_tok.astype(f32)
    fs = feat_srl.astype(f32)
    wnt = W_node_trans.T.astype(f32)
    wa1 = W_node_att[:, :D].T.astype(f32)
    wa2 = W_node_att[:, D:].T.astype(f32)
    wih = W_ih.T.astype(f32)
    whh = W_hh.T.astype(f32)
    bnt = b_node_trans.reshape(1, D).astype(f32)
    batt = b_node_att.reshape(1, D).astype(f32)
    bih = b_ih.reshape(1, 3 * D).astype(f32)
    bhh = b_hh.reshape(1, 3 * D).astype(f32)
    idx2_a = edge_tok2srl.astype(jnp.int32).reshape(2, NCHT, CH)
    idx2_a = jnp.swapaxes(idx2_a, 0, 1)
    idx2_b = edge_srl2tok.astype(jnp.int32).reshape(2, NCHT, CH)
    idx2_b = jnp.swapaxes(idx2_b, 0, 1)

    src_tab, dst_tab, ght_tab, _, mxo = _stage1(
        ft, fs, wnt, wa1, wa2, whh, bnt, batt, bhh)
    acc_a = _pass_a(src_tab.reshape(NC * N, 2 * HF), dst_tab,
                    mxo.reshape(D), idx2_a)
    h_srl, git_tab = _stage3(acc_a.reshape(NC, N, 2 * HF), fs, wih, bih)
    acc_b = _pass_b(git_tab.reshape(NC * N, 4 * HF),
                    ght_tab.reshape(NC * N, 4 * HF), idx2_b)
    acc_b = acc_b.reshape(NC, N, HF)
    h_at = jnp.concatenate([acc_b[0], acc_b[1]], axis=1)
    h_tok = _stage5(h_at, wih, whh, bih, bhh)
    return (h_tok, h_srl)
